# baseline (speedup 1.0000x reference)
import functools

import jax
import jax.numpy as jnp
from jax import lax
from jax.experimental import pallas as pl
from jax.experimental.pallas import tpu as pltpu
from jax.experimental.pallas import tpu_sc as plsc

MAX_DIST = 20
NUM_HEADS = 16

_NC = 2
_NS = 16
_NW = _NC * _NS
_LANES = 16

_CHUNK = 2048
_STREAM = 1024
_GSLICE = 512
_NBUF = 2


def _sc_lookup(dist_hbm, table_hbm, out_hbm, dist_v, idx_v, rows_v, tab_v,
               tab_sp, isem0, isem1, osem0, osem1, gsem):
    b = dist_hbm.shape[0]
    b_per_w = b // _NW
    n_chunks = b_per_w // _CHUNK
    wid = lax.axis_index("s") * _NC + lax.axis_index("c")
    base = wid * b_per_w
    isems = (isem0, isem1)
    osems = (osem0, osem1)

    pltpu.sync_copy(table_hbm, tab_v)

    @pl.when(lax.axis_index("s") == 0)
    def _stage_table():
        pltpu.sync_copy(table_hbm, tab_sp)

    plsc.subcore_barrier()

    iota16 = lax.iota(jnp.int32, _LANES)
    dnums = lax.GatherDimensionNumbers(
        offset_dims=(), collapsed_slice_dims=(0,), start_index_map=(0,))

    def lane_broadcast(v, k):
        return lax.gather(
            v, jnp.full((_LANES, 1), k, jnp.int32), dnums, (1,),
            mode=lax.GatherScatterMode.PROMISE_IN_BOUNDS)

    def in_copy(t, bi):
        return pltpu.make_async_copy(
            dist_hbm.at[pl.ds(base + t * _CHUNK, _CHUNK)], dist_v.at[bi],
            isems[bi])

    def out_copy(t, bi):
        return pltpu.make_async_copy(
            rows_v.at[bi], out_hbm.at[pl.ds(base + t * _CHUNK, _CHUNK)],
            osems[bi])

    in_copy(0, 0).start()
    in_copy(1, 1).start()

    @pl.loop(0, n_chunks, step=_NBUF)
    def _chunk_pair(t0):
        for bi in range(_NBUF):
            t = t0 + bi
            in_copy(t, bi).wait()

            @pl.when(t >= _NBUF)
            def _drain_prev_writeback():
                out_copy(t - _NBUF, bi).wait()

            @plsc.parallel_loop(0, _STREAM // _LANES, unroll=8)
            def _clamp(j):
                v = dist_v[bi, pl.ds(j * _LANES, _LANES)]
                idx_v[bi, pl.ds(j * _LANES, _LANES)] = jnp.clip(
                    v + 1, 0, MAX_DIST + 1)

            copies = []
            for j in range(_STREAM // _GSLICE):
                copies.append(
                    pltpu.make_async_copy(
                        tab_sp.at[idx_v.at[bi].at[pl.ds(j * _GSLICE, _GSLICE)]],
                        rows_v.at[bi].at[pl.ds(j * _GSLICE, _GSLICE)],
                        gsem,
                    )
                )
            for c in copies:
                c.start()

            @plsc.parallel_loop(_STREAM // _LANES, _CHUNK // _LANES, unroll=4)
            def _group(g):
                v = dist_v[bi, pl.ds(g * _LANES, _LANES)]
                v = jnp.clip(v + 1, 0, MAX_DIST + 1)
                for k in range(_LANES):
                    bvec = lane_broadcast(v, k)
                    row = plsc.load_gather(tab_v, [bvec, iota16])
                    rows_v[bi, g * _LANES + k] = row

            for c in copies:
                c.wait()

            @pl.when(t + _NBUF < n_chunks)
            def _prefetch_next():
                in_copy(t + _NBUF, bi).start()

            out_copy(t, bi).start()

    out_copy(n_chunks - 2, 0).wait()
    out_copy(n_chunks - 1, 1).wait()


def kernel(dist, table):
    b = dist.size
    flat = dist.reshape((b,)).astype(jnp.int32)
    run = functools.partial(
        pl.kernel,
        out_type=jax.ShapeDtypeStruct((b, NUM_HEADS), jnp.float32),
        mesh=plsc.VectorSubcoreMesh(
            core_axis_name="c", subcore_axis_name="s",
            num_cores=_NC, num_subcores=_NS),
        scratch_types=[
            pltpu.VMEM((_NBUF, _CHUNK), jnp.int32),
            pltpu.VMEM((_NBUF, _STREAM), jnp.int32),
            pltpu.VMEM((_NBUF, _CHUNK, NUM_HEADS), jnp.float32),
            pltpu.VMEM((MAX_DIST + 2, NUM_HEADS), jnp.float32),
            pltpu.VMEM_SHARED((MAX_DIST + 2, NUM_HEADS), jnp.float32),
            pltpu.SemaphoreType.DMA,
            pltpu.SemaphoreType.DMA,
            pltpu.SemaphoreType.DMA,
            pltpu.SemaphoreType.DMA,
            pltpu.SemaphoreType.DMA,
        ],
        compiler_params=pltpu.CompilerParams(
            use_tc_tiling_on_sc=False, needs_layout_passes=False),
    )(_sc_lookup)
    out = run(flat, table)
    return out.reshape(dist.shape + (NUM_HEADS,))


# speedup vs baseline: 1.0545x; 1.0545x over previous
import functools

import jax
import jax.numpy as jnp
from jax import lax
from jax.experimental import pallas as pl
from jax.experimental.pallas import tpu as pltpu
from jax.experimental.pallas import tpu_sc as plsc

MAX_DIST = 20
NUM_HEADS = 16

_NC = 2
_NS = 16
_NW = _NC * _NS
_LANES = 16

_CHUNK = 2048
_NBUF = 2


def _sc_lookup(dist_hbm, table_hbm, out_hbm, dist_v, rows_v, tab_v,
               isem0, isem1, osem0, osem1):
    b = dist_hbm.shape[0]
    b_per_w = b // _NW
    n_chunks = b_per_w // _CHUNK
    wid = lax.axis_index("s") * _NC + lax.axis_index("c")
    base = wid * b_per_w
    isems = (isem0, isem1)
    osems = (osem0, osem1)

    pltpu.sync_copy(table_hbm, tab_v)

    iota16 = lax.iota(jnp.int32, _LANES)
    dnums = lax.GatherDimensionNumbers(
        offset_dims=(), collapsed_slice_dims=(0,), start_index_map=(0,))

    def lane_broadcast(v, k):
        return lax.gather(
            v, jnp.full((_LANES, 1), k, jnp.int32), dnums, (1,),
            mode=lax.GatherScatterMode.PROMISE_IN_BOUNDS)

    def in_copy(t, bi):
        return pltpu.make_async_copy(
            dist_hbm.at[pl.ds(base + t * _CHUNK, _CHUNK)], dist_v.at[bi],
            isems[bi])

    def out_copy(t, bi):
        return pltpu.make_async_copy(
            rows_v.at[bi],
            out_hbm.at[pl.ds((base + t * _CHUNK) * NUM_HEADS,
                             _CHUNK * NUM_HEADS)],
            osems[bi])

    in_copy(0, 0).start()
    in_copy(1, 1).start()

    @pl.loop(0, n_chunks, step=_NBUF)
    def _chunk_pair(t0):
        for bi in range(_NBUF):
            t = t0 + bi
            in_copy(t, bi).wait()

            @pl.when(t >= _NBUF)
            def _drain_prev_writeback():
                out_copy(t - _NBUF, bi).wait()

            @plsc.parallel_loop(0, _CHUNK // _LANES, unroll=4)
            def _group(g):
                v = dist_v[bi, pl.ds(g * _LANES, _LANES)]
                v = jnp.clip(v + 1, 0, MAX_DIST + 1)
                rbase = g * (_LANES * NUM_HEADS)
                for k in range(_LANES):
                    bvec = lane_broadcast(v, k)
                    row = plsc.load_gather(tab_v, [bvec, iota16])
                    rows_v[bi, pl.ds(rbase + k * NUM_HEADS, NUM_HEADS)] = row

            @pl.when(t + _NBUF < n_chunks)
            def _prefetch_next():
                in_copy(t + _NBUF, bi).start()

            out_copy(t, bi).start()

    out_copy(n_chunks - 2, 0).wait()
    out_copy(n_chunks - 1, 1).wait()


def kernel(dist, table):
    b = dist.size
    flat = dist.reshape((b,)).astype(jnp.int32)
    run = functools.partial(
        pl.kernel,
        out_type=jax.ShapeDtypeStruct((b * NUM_HEADS,), jnp.float32),
        mesh=plsc.VectorSubcoreMesh(
            core_axis_name="c", subcore_axis_name="s",
            num_cores=_NC, num_subcores=_NS),
        scratch_types=[
            pltpu.VMEM((_NBUF, _CHUNK), jnp.int32),
            pltpu.VMEM((_NBUF, _CHUNK * NUM_HEADS), jnp.float32),
            pltpu.VMEM((MAX_DIST + 2, NUM_HEADS), jnp.float32),
            pltpu.SemaphoreType.DMA,
            pltpu.SemaphoreType.DMA,
            pltpu.SemaphoreType.DMA,
            pltpu.SemaphoreType.DMA,
        ],
        compiler_params=pltpu.CompilerParams(
            use_tc_tiling_on_sc=False, needs_layout_passes=False),
    )(_sc_lookup)
    out = run(flat, table)
    return out.reshape(dist.shape + (NUM_HEADS,))
